# baseline (speedup 1.0000x reference)
import functools

import jax
import jax.numpy as jnp
from jax import lax
from jax.experimental import pallas as pl
from jax.experimental.pallas import tpu as pltpu
from jax.experimental.pallas import tpu_sc as plsc

N = 10000
NP = 10240
E = 320000
NC = 2
NS = 16
NW = NC * NS
EB = 32
NLANE = 8
NH = 8
EJH = 40
EPT = NH * EJH * EB
E_PAD = NW * EPT
ROWS_PER_TILE = NP // NS
ZB = 32

_MESH = plsc.VectorSubcoreMesh(core_axis_name="c", subcore_axis_name="s")


def _fill_const(ref, rows, width, value):
    v = jnp.full((16,), value, jnp.float32)

    def body(j, carry):
        for k in range(width // 16):
            ref[j, pl.ds(k * 16, 16)] = v
        return carry

    lax.fori_loop(0, rows, body, 0)


def _zero_slice(zbuf, acc, r0, zsem):
    nz = ROWS_PER_TILE // ZB
    for z in range(nz):
        pltpu.async_copy(zbuf.at[pl.ds(0, ZB)], acc.at[pl.ds(r0 + z * ZB, ZB)], zsem)
    for z in range(nz):
        pltpu.make_async_copy(zbuf.at[pl.ds(0, ZB)], acc.at[pl.ds(r0, ZB)], zsem).wait()


def _pad_edges(edge):
    pad = E_PAD - E
    ar = jnp.arange(pad, dtype=edge.dtype)
    src = jnp.concatenate([edge[0], ar % N])
    dst = jnp.concatenate([edge[1], N + ar % (NP - N)])
    return (src.reshape(NW, NH, EJH, EB), dst.reshape(NW, NH, EJH, EB))


def _deg_sc(dst4):
    LAG = 8

    @functools.partial(
        pl.kernel,
        out_type=jax.ShapeDtypeStruct((NC, NP, 128), jnp.float32),
        mesh=_MESH,
        scratch_types=[
            pltpu.VMEM((EJH, EB), jnp.int32),
            pltpu.VMEM((EB, 128), jnp.float32),
            pltpu.VMEM((ZB, 128), jnp.float32),
            pltpu.SemaphoreType.DMA,
            pltpu.SemaphoreType.DMA,
            pltpu.VMEM_SHARED((NP, 128), jnp.float32),
        ],
    )
    def k(dst_hbm, out_hbm, idx_v, ones_v, zero_v, ssem, zsem, acc):
        c = lax.axis_index("c")
        s = lax.axis_index("s")
        w = c * NS + s
        r0 = s * ROWS_PER_TILE
        _fill_const(ones_v, EB, 128, 1.0)
        _fill_const(zero_v, ZB, 128, 0.0)
        _zero_slice(zero_v, acc, r0, zsem)
        plsc.subcore_barrier()
        for half in range(NH):
            pltpu.sync_copy(dst_hbm.at[w, half], idx_v)

            def body(j, carry):
                pltpu.async_copy(ones_v, acc.at[idx_v.at[j]], ssem, add=True)

                @pl.when(j >= LAG)
                def _():
                    pltpu.make_async_copy(ones_v, acc.at[idx_v.at[j]], ssem).wait()

                return carry

            lax.fori_loop(0, EJH, body, 0)
            for _ in range(LAG):
                pltpu.make_async_copy(ones_v, acc.at[idx_v.at[0]], ssem).wait()
        plsc.subcore_barrier()
        pltpu.sync_copy(acc.at[pl.ds(r0, ROWS_PER_TILE)],
                        out_hbm.at[c, pl.ds(r0, ROWS_PER_TILE)])

    return k(dst4)


def _prop_sc(tables, src4, dst4):
    C = len(tables)

    @functools.partial(
        pl.kernel,
        out_type=[jax.ShapeDtypeStruct((NC, NP, 128), jnp.float32) for _ in range(C)],
        mesh=_MESH,
        scratch_types=[
            pltpu.VMEM((EJH, EB), jnp.int32),
            pltpu.VMEM((EJH, EB), jnp.int32),
        ] + [pltpu.VMEM((EB, 128), jnp.float32) for _ in range(NLANE)]
        + [pltpu.SemaphoreType.DMA for _ in range(2 * NLANE + 1)]
        + [pltpu.VMEM_SHARED((NP, 128), jnp.float32)],
    )
    def k(*refs):
        h_hbms = refs[:C]
        src_hbm, dst_hbm = refs[C], refs[C + 1]
        outs = refs[C + 2:C + 2 + C]
        rest = refs[C + 2 + C:]
        src_v, dst_v = rest[0], rest[1]
        bufs = rest[2:2 + NLANE]
        gs = rest[2 + NLANE:2 + 2 * NLANE]
        ss = rest[2 + 2 * NLANE:2 + 3 * NLANE]
        zsem = rest[2 + 3 * NLANE]
        acc = rest[2 + 3 * NLANE + 1]
        c = lax.axis_index("c")
        s = lax.axis_index("s")
        w = c * NS + s
        r0 = s * ROWS_PER_TILE
        for ci in range(C):
            h = h_hbms[ci]
            _fill_const(bufs[0], ZB, 128, 0.0)
            _zero_slice(bufs[0], acc, r0, zsem)
            plsc.subcore_barrier()
            for half in range(NH):
                pltpu.sync_copy(src_hbm.at[w, half], src_v)
                pltpu.sync_copy(dst_hbm.at[w, half], dst_v)
                for k_ in range(NLANE):
                    pltpu.async_copy(h.at[src_v.at[k_]], bufs[k_], gs[k_])

                def body(gidx, carry, h=h):
                    base = gidx * NLANE
                    for k_ in range(NLANE):
                        j = base + k_
                        pltpu.make_async_copy(h.at[src_v.at[j]], bufs[k_], gs[k_]).wait()
                        pltpu.async_copy(bufs[k_], acc.at[dst_v.at[j]], ss[k_], add=True)
                    for k_ in range(NLANE):
                        j = base + k_

                        @pl.when(j + NLANE < EJH)
                        def _(j=j, k_=k_):
                            pltpu.make_async_copy(bufs[k_], acc.at[dst_v.at[j]], ss[k_]).wait()
                            pltpu.async_copy(h.at[src_v.at[j + NLANE]], bufs[k_], gs[k_])

                    return carry

                lax.fori_loop(0, EJH // NLANE, body, 0)
                for k_ in range(NLANE):
                    pltpu.make_async_copy(bufs[k_], acc.at[dst_v.at[k_]], ss[k_]).wait()
            plsc.subcore_barrier()
            pltpu.sync_copy(acc.at[pl.ds(r0, ROWS_PER_TILE)],
                            outs[ci].at[c, pl.ds(r0, ROWS_PER_TILE)])

    return k(*tables, src4, dst4)


BN = 400
GRID = (N // BN,)


def _rowspec(*lead):
    nl = len(lead)
    return pl.BlockSpec(tuple(lead) + (BN, 128),
                        lambda i, nl=nl: (0,) * nl + (i, 0))


def _fullspec(shape):
    nd = len(shape)
    return pl.BlockSpec(shape, lambda i, nd=nd: (0,) * nd)


def _layer_norm(z, g, b):
    mu = jnp.mean(z, axis=-1, keepdims=True)
    var = jnp.mean((z - mu) ** 2, axis=-1, keepdims=True)
    return (z - mu) * lax.rsqrt(var + 1e-5) * g + b


def _prep_tc(degp, x):
    def body(deg_ref, x_ref, dinv_ref, xp_ref):
        d = deg_ref[0] + deg_ref[1] + 1.0
        dvb = lax.rsqrt(d)
        dinv_ref[...] = dvb
        xp_ref[...] = x_ref[...] * dvb

    return pl.pallas_call(
        body,
        grid=GRID,
        in_specs=[_rowspec(NC), _rowspec()],
        out_specs=[_rowspec(), _rowspec()],
        out_shape=[jax.ShapeDtypeStruct((N, 128), jnp.float32)] * 2,
    )(degp, x)


def _layer1_tc(S1, xp, dinv, W1, b1, g1, bt1):
    def body(S_ref, xp_ref, dv_ref, W_ref, b_ref, g_ref, bt_ref, *out_refs):
        u = (S_ref[0] + S_ref[1] + xp_ref[...]) * dv_ref[...]
        z = jnp.dot(u, W_ref[...], preferred_element_type=jnp.float32)
        z = jax.nn.relu(z + b_ref[...])
        y = _layer_norm(z, g_ref[...], bt_ref[...])
        for ci in range(8):
            out_refs[ci][...] = y[:, ci * 128:(ci + 1) * 128]

    return pl.pallas_call(
        body,
        grid=GRID,
        in_specs=[_rowspec(NC), _rowspec(), _rowspec(),
                  _fullspec((128, 1024)), _fullspec((1, 1024)),
                  _fullspec((1, 1024)), _fullspec((1, 1024))],
        out_specs=[_rowspec()] * 8,
        out_shape=[jax.ShapeDtypeStruct((N, 128), jnp.float32)] * 8,
    )(S1, xp, dinv, W1, b1, g1, bt1)


def _matmul_tc(x_chunks, Wr, dinv):
    CI = len(x_chunks)
    CO = Wr.shape[2] // 128

    def body(*refs):
        x_refs = refs[:CI]
        W_ref, dv_ref = refs[CI], refs[CI + 1]
        out_refs = refs[CI + 2:]
        acc = jnp.zeros((BN, Wr.shape[2]), jnp.float32)
        for ci in range(CI):
            acc = acc + jnp.dot(x_refs[ci][...], W_ref[ci],
                                preferred_element_type=jnp.float32)
        z = acc * dv_ref[:, 0:1]
        for co in range(CO):
            out_refs[co][...] = z[:, co * 128:(co + 1) * 128]

    return pl.pallas_call(
        body,
        grid=GRID,
        in_specs=[_rowspec()] * CI + [_fullspec(Wr.shape), _rowspec()],
        out_specs=[_rowspec()] * CO,
        out_shape=[jax.ShapeDtypeStruct((N, 128), jnp.float32)] * CO,
    )(*x_chunks, Wr, dinv)


def _epilogue_tc(S_parts, h_chunks, dinv, b, g, bt, chunk_major_out):
    C = len(h_chunks)
    D = C * 128

    def body(*refs):
        S_refs = refs[:C]
        h_refs = refs[C:2 * C]
        dv_ref, b_ref, g_ref, bt_ref = refs[2 * C:2 * C + 4]
        out_refs = refs[2 * C + 4:]
        dv = dv_ref[:, 0:1]
        parts = []
        for ci in range(C):
            zc = (S_refs[ci][0] + S_refs[ci][1] + h_refs[ci][...]) * dv
            parts.append(zc + b_ref[:, ci * 128:(ci + 1) * 128])
        z = jax.nn.relu(jnp.concatenate(parts, axis=1))
        y = _layer_norm(z, g_ref[...], bt_ref[...])
        if chunk_major_out:
            for ci in range(C):
                out_refs[ci][...] = y[:, ci * 128:(ci + 1) * 128]
        else:
            out_refs[0][...] = y

    if chunk_major_out:
        out_specs = [_rowspec()] * C
        out_shape = [jax.ShapeDtypeStruct((N, 128), jnp.float32)] * C
    else:
        out_specs = [pl.BlockSpec((BN, D), lambda i: (i, 0))]
        out_shape = [jax.ShapeDtypeStruct((N, D), jnp.float32)]
    res = pl.pallas_call(
        body,
        grid=GRID,
        in_specs=[_rowspec(NC)] * C + [_rowspec()] * C
                 + [_rowspec(), _fullspec((1, D)), _fullspec((1, D)),
                    _fullspec((1, D))],
        out_specs=out_specs,
        out_shape=out_shape,
    )(*S_parts, *h_chunks, dinv, b, g, bt)
    return res


def kernel(data_batch, edge, W1, b1, g1, bt1, W2, b2, g2, bt2, W3, b3, g3, bt3):
    src4, dst4 = _pad_edges(edge)

    degp = _deg_sc(dst4)
    dinv, xp = _prep_tc(degp, data_batch)

    (S1,) = _prop_sc([xp], src4, dst4)
    x1_chunks = _layer1_tc(S1, xp, dinv, W1, b1.reshape(1, -1),
                           g1.reshape(1, -1), bt1.reshape(1, -1))

    h2_chunks = _matmul_tc(x1_chunks, W2.reshape(8, 128, 512), dinv)
    S2_parts = _prop_sc(list(h2_chunks), src4, dst4)
    x2_chunks = _epilogue_tc(S2_parts, h2_chunks, dinv, b2.reshape(1, -1),
                             g2.reshape(1, -1), bt2.reshape(1, -1), True)

    h3_chunks = _matmul_tc(x2_chunks, W3.reshape(4, 128, 256), dinv)
    S3_parts = _prop_sc(list(h3_chunks), src4, dst4)
    (x3,) = _epilogue_tc(S3_parts, h3_chunks, dinv, b3.reshape(1, -1),
                         g3.reshape(1, -1), bt3.reshape(1, -1), False)
    return x3


# speedup vs baseline: 1.0590x; 1.0590x over previous
import functools

import jax
import jax.numpy as jnp
from jax import lax
from jax.experimental import pallas as pl
from jax.experimental.pallas import tpu as pltpu
from jax.experimental.pallas import tpu_sc as plsc

N = 10000
NP = 10240
E = 320000
NC = 2
NS = 16
NW = NC * NS
EB = 64
NLANE = 4
NH = 4
EJH = 40
EPT = NH * EJH * EB
E_PAD = NW * EPT
ROWS_PER_TILE = NP // NS
ZB = 64

_MESH = plsc.VectorSubcoreMesh(core_axis_name="c", subcore_axis_name="s")


def _fill_const(ref, rows, width, value):
    v = jnp.full((16,), value, jnp.float32)

    def body(j, carry):
        for k in range(width // 16):
            ref[j, pl.ds(k * 16, 16)] = v
        return carry

    lax.fori_loop(0, rows, body, 0)


def _zero_slice(zbuf, acc, r0, zsem):
    nz = ROWS_PER_TILE // ZB
    for z in range(nz):
        pltpu.async_copy(zbuf.at[pl.ds(0, ZB)], acc.at[pl.ds(r0 + z * ZB, ZB)], zsem)
    for z in range(nz):
        pltpu.make_async_copy(zbuf.at[pl.ds(0, ZB)], acc.at[pl.ds(r0, ZB)], zsem).wait()


def _pad_edges(edge):
    pad = E_PAD - E
    ar = jnp.arange(pad, dtype=edge.dtype)
    src = jnp.concatenate([edge[0], ar % N])
    dst = jnp.concatenate([edge[1], N + ar % (NP - N)])
    return (src.reshape(NW, NH, EJH, EB), dst.reshape(NW, NH, EJH, EB))


def _deg_sc(dst4):
    LAG = 8

    @functools.partial(
        pl.kernel,
        out_type=jax.ShapeDtypeStruct((NC, NP, 128), jnp.float32),
        mesh=_MESH,
        scratch_types=[
            pltpu.VMEM((EJH, EB), jnp.int32),
            pltpu.VMEM((EB, 128), jnp.float32),
            pltpu.VMEM((ZB, 128), jnp.float32),
            pltpu.SemaphoreType.DMA,
            pltpu.SemaphoreType.DMA,
            pltpu.VMEM_SHARED((NP, 128), jnp.float32),
        ],
    )
    def k(dst_hbm, out_hbm, idx_v, ones_v, zero_v, ssem, zsem, acc):
        c = lax.axis_index("c")
        s = lax.axis_index("s")
        w = c * NS + s
        r0 = s * ROWS_PER_TILE
        _fill_const(ones_v, EB, 128, 1.0)
        _fill_const(zero_v, ZB, 128, 0.0)
        _zero_slice(zero_v, acc, r0, zsem)
        plsc.subcore_barrier()
        for half in range(NH):
            pltpu.sync_copy(dst_hbm.at[w, half], idx_v)

            def body(j, carry):
                pltpu.async_copy(ones_v, acc.at[idx_v.at[j]], ssem, add=True)

                @pl.when(j >= LAG)
                def _():
                    pltpu.make_async_copy(ones_v, acc.at[idx_v.at[j]], ssem).wait()

                return carry

            lax.fori_loop(0, EJH, body, 0)
            for _ in range(LAG):
                pltpu.make_async_copy(ones_v, acc.at[idx_v.at[0]], ssem).wait()
        plsc.subcore_barrier()
        pltpu.sync_copy(acc.at[pl.ds(r0, ROWS_PER_TILE)],
                        out_hbm.at[c, pl.ds(r0, ROWS_PER_TILE)])

    return k(dst4)


def _prop_sc(tables, src4, dst4):
    C = len(tables)

    @functools.partial(
        pl.kernel,
        out_type=[jax.ShapeDtypeStruct((NC, NP, 128), jnp.float32) for _ in range(C)],
        mesh=_MESH,
        scratch_types=[
            pltpu.VMEM((EJH, EB), jnp.int32),
            pltpu.VMEM((EJH, EB), jnp.int32),
        ] + [pltpu.VMEM((EB, 128), jnp.float32) for _ in range(NLANE)]
        + [pltpu.SemaphoreType.DMA for _ in range(2 * NLANE + 1)]
        + [pltpu.VMEM_SHARED((NP, 128), jnp.float32)],
    )
    def k(*refs):
        h_hbms = refs[:C]
        src_hbm, dst_hbm = refs[C], refs[C + 1]
        outs = refs[C + 2:C + 2 + C]
        rest = refs[C + 2 + C:]
        src_v, dst_v = rest[0], rest[1]
        bufs = rest[2:2 + NLANE]
        gs = rest[2 + NLANE:2 + 2 * NLANE]
        ss = rest[2 + 2 * NLANE:2 + 3 * NLANE]
        zsem = rest[2 + 3 * NLANE]
        acc = rest[2 + 3 * NLANE + 1]
        c = lax.axis_index("c")
        s = lax.axis_index("s")
        w = c * NS + s
        r0 = s * ROWS_PER_TILE
        for ci in range(C):
            h = h_hbms[ci]
            _fill_const(bufs[0], ZB, 128, 0.0)
            _zero_slice(bufs[0], acc, r0, zsem)
            plsc.subcore_barrier()
            for half in range(NH):
                pltpu.sync_copy(src_hbm.at[w, half], src_v)
                pltpu.sync_copy(dst_hbm.at[w, half], dst_v)
                for k_ in range(NLANE):
                    pltpu.async_copy(h.at[src_v.at[k_]], bufs[k_], gs[k_])

                def body(gidx, carry, h=h):
                    base = gidx * NLANE
                    for k_ in range(NLANE):
                        j = base + k_
                        pltpu.make_async_copy(h.at[src_v.at[j]], bufs[k_], gs[k_]).wait()
                        pltpu.async_copy(bufs[k_], acc.at[dst_v.at[j]], ss[k_], add=True)
                    for k_ in range(NLANE):
                        j = base + k_

                        @pl.when(j + NLANE < EJH)
                        def _(j=j, k_=k_):
                            pltpu.make_async_copy(bufs[k_], acc.at[dst_v.at[j]], ss[k_]).wait()
                            pltpu.async_copy(h.at[src_v.at[j + NLANE]], bufs[k_], gs[k_])

                    return carry

                lax.fori_loop(0, EJH // NLANE, body, 0)
                for k_ in range(NLANE):
                    pltpu.make_async_copy(bufs[k_], acc.at[dst_v.at[k_]], ss[k_]).wait()
            plsc.subcore_barrier()
            pltpu.sync_copy(acc.at[pl.ds(r0, ROWS_PER_TILE)],
                            outs[ci].at[c, pl.ds(r0, ROWS_PER_TILE)])

    return k(*tables, src4, dst4)


BN = 400
GRID = (N // BN,)


def _rowspec(*lead):
    nl = len(lead)
    return pl.BlockSpec(tuple(lead) + (BN, 128),
                        lambda i, nl=nl: (0,) * nl + (i, 0))


def _fullspec(shape):
    nd = len(shape)
    return pl.BlockSpec(shape, lambda i, nd=nd: (0,) * nd)


def _layer_norm(z, g, b):
    mu = jnp.mean(z, axis=-1, keepdims=True)
    var = jnp.mean((z - mu) ** 2, axis=-1, keepdims=True)
    return (z - mu) * lax.rsqrt(var + 1e-5) * g + b


def _prep_tc(degp, x):
    def body(deg_ref, x_ref, dinv_ref, xp_ref):
        d = deg_ref[0] + deg_ref[1] + 1.0
        dvb = lax.rsqrt(d)
        dinv_ref[...] = dvb
        xp_ref[...] = x_ref[...] * dvb

    return pl.pallas_call(
        body,
        grid=GRID,
        in_specs=[_rowspec(NC), _rowspec()],
        out_specs=[_rowspec(), _rowspec()],
        out_shape=[jax.ShapeDtypeStruct((N, 128), jnp.float32)] * 2,
    )(degp, x)


def _layer1_tc(S1, xp, dinv, W1, b1, g1, bt1):
    def body(S_ref, xp_ref, dv_ref, W_ref, b_ref, g_ref, bt_ref, *out_refs):
        u = (S_ref[0] + S_ref[1] + xp_ref[...]) * dv_ref[...]
        z = jnp.dot(u.astype(jnp.bfloat16), W_ref[...].astype(jnp.bfloat16),
                    preferred_element_type=jnp.float32)
        z = jax.nn.relu(z + b_ref[...])
        y = _layer_norm(z, g_ref[...], bt_ref[...])
        for ci in range(8):
            out_refs[ci][...] = y[:, ci * 128:(ci + 1) * 128]

    return pl.pallas_call(
        body,
        grid=GRID,
        in_specs=[_rowspec(NC), _rowspec(), _rowspec(),
                  _fullspec((128, 1024)), _fullspec((1, 1024)),
                  _fullspec((1, 1024)), _fullspec((1, 1024))],
        out_specs=[_rowspec()] * 8,
        out_shape=[jax.ShapeDtypeStruct((N, 128), jnp.float32)] * 8,
    )(S1, xp, dinv, W1, b1, g1, bt1)


def _matmul_tc(x_chunks, Wr, dinv):
    CI = len(x_chunks)
    CO = Wr.shape[2] // 128

    def body(*refs):
        x_refs = refs[:CI]
        W_ref, dv_ref = refs[CI], refs[CI + 1]
        out_refs = refs[CI + 2:]
        acc = jnp.zeros((BN, Wr.shape[2]), jnp.float32)
        for ci in range(CI):
            acc = acc + jnp.dot(x_refs[ci][...].astype(jnp.bfloat16),
                                W_ref[ci].astype(jnp.bfloat16),
                                preferred_element_type=jnp.float32)
        z = acc * dv_ref[:, 0:1]
        for co in range(CO):
            out_refs[co][...] = z[:, co * 128:(co + 1) * 128]

    return pl.pallas_call(
        body,
        grid=GRID,
        in_specs=[_rowspec()] * CI + [_fullspec(Wr.shape), _rowspec()],
        out_specs=[_rowspec()] * CO,
        out_shape=[jax.ShapeDtypeStruct((N, 128), jnp.float32)] * CO,
    )(*x_chunks, Wr, dinv)


def _epilogue_tc(S_parts, h_chunks, dinv, b, g, bt, chunk_major_out):
    C = len(h_chunks)
    D = C * 128

    def body(*refs):
        S_refs = refs[:C]
        h_refs = refs[C:2 * C]
        dv_ref, b_ref, g_ref, bt_ref = refs[2 * C:2 * C + 4]
        out_refs = refs[2 * C + 4:]
        dv = dv_ref[:, 0:1]
        parts = []
        for ci in range(C):
            zc = (S_refs[ci][0] + S_refs[ci][1] + h_refs[ci][...]) * dv
            parts.append(zc + b_ref[:, ci * 128:(ci + 1) * 128])
        z = jax.nn.relu(jnp.concatenate(parts, axis=1))
        y = _layer_norm(z, g_ref[...], bt_ref[...])
        if chunk_major_out:
            for ci in range(C):
                out_refs[ci][...] = y[:, ci * 128:(ci + 1) * 128]
        else:
            out_refs[0][...] = y

    if chunk_major_out:
        out_specs = [_rowspec()] * C
        out_shape = [jax.ShapeDtypeStruct((N, 128), jnp.float32)] * C
    else:
        out_specs = [pl.BlockSpec((BN, D), lambda i: (i, 0))]
        out_shape = [jax.ShapeDtypeStruct((N, D), jnp.float32)]
    res = pl.pallas_call(
        body,
        grid=GRID,
        in_specs=[_rowspec(NC)] * C + [_rowspec()] * C
                 + [_rowspec(), _fullspec((1, D)), _fullspec((1, D)),
                    _fullspec((1, D))],
        out_specs=out_specs,
        out_shape=out_shape,
    )(*S_parts, *h_chunks, dinv, b, g, bt)
    return res


def kernel(data_batch, edge, W1, b1, g1, bt1, W2, b2, g2, bt2, W3, b3, g3, bt3):
    src4, dst4 = _pad_edges(edge)

    degp = _deg_sc(dst4)
    dinv, xp = _prep_tc(degp, data_batch)

    (S1,) = _prop_sc([xp], src4, dst4)
    x1_chunks = _layer1_tc(S1, xp, dinv, W1, b1.reshape(1, -1),
                           g1.reshape(1, -1), bt1.reshape(1, -1))

    h2_chunks = _matmul_tc(x1_chunks, W2.reshape(8, 128, 512), dinv)
    S2_parts = _prop_sc(list(h2_chunks), src4, dst4)
    x2_chunks = _epilogue_tc(S2_parts, h2_chunks, dinv, b2.reshape(1, -1),
                             g2.reshape(1, -1), bt2.reshape(1, -1), True)

    h3_chunks = _matmul_tc(x2_chunks, W3.reshape(4, 128, 256), dinv)
    S3_parts = _prop_sc(list(h3_chunks), src4, dst4)
    (x3,) = _epilogue_tc(S3_parts, h3_chunks, dinv, b3.reshape(1, -1),
                         g3.reshape(1, -1), bt3.reshape(1, -1), False)
    return x3


# speedup vs baseline: 1.1154x; 1.0533x over previous
import functools

import jax
import jax.numpy as jnp
from jax import lax
from jax.experimental import pallas as pl
from jax.experimental.pallas import tpu as pltpu
from jax.experimental.pallas import tpu_sc as plsc

N = 10000
NP = 10240
E = 320000
NC = 2
NS = 16
NW = NC * NS
EB = 64
NLANE = 4
NH = 4
EJH = 40
EPT = NH * EJH * EB
E_PAD = NW * EPT
ROWS_PER_TILE = NP // NS
ZB = 64

_MESH = plsc.VectorSubcoreMesh(core_axis_name="c", subcore_axis_name="s")


def _fill_const(ref, rows, width, value):
    v = jnp.full((16,), value, jnp.float32)

    def body(j, carry):
        for k in range(width // 16):
            ref[j, pl.ds(k * 16, 16)] = v
        return carry

    lax.fori_loop(0, rows, body, 0)


def _zero_slice(zbuf, acc, r0, zsem):
    nz = ROWS_PER_TILE // ZB
    for z in range(nz):
        pltpu.async_copy(zbuf.at[pl.ds(0, ZB)], acc.at[pl.ds(r0 + z * ZB, ZB)], zsem)
    for z in range(nz):
        pltpu.make_async_copy(zbuf.at[pl.ds(0, ZB)], acc.at[pl.ds(r0, ZB)], zsem).wait()


def _pad_edges(edge):
    pad = E_PAD - E
    ar = jnp.arange(pad, dtype=edge.dtype)
    src = jnp.concatenate([edge[0], ar % N])
    dst = jnp.concatenate([edge[1], N + ar % (NP - N)])
    return (src.reshape(NW, NH, EJH, EB), dst.reshape(NW, NH, EJH, EB))


def _deg_sc(dst4):
    LAG = 8

    @functools.partial(
        pl.kernel,
        out_type=jax.ShapeDtypeStruct((NC, NP, 128), jnp.float32),
        mesh=_MESH,
        scratch_types=[
            pltpu.VMEM((EJH, EB), jnp.int32),
            pltpu.VMEM((EB, 128), jnp.float32),
            pltpu.VMEM((ZB, 128), jnp.float32),
            pltpu.SemaphoreType.DMA,
            pltpu.SemaphoreType.DMA,
            pltpu.VMEM_SHARED((NP, 128), jnp.float32),
        ],
    )
    def k(dst_hbm, out_hbm, idx_v, ones_v, zero_v, ssem, zsem, acc):
        c = lax.axis_index("c")
        s = lax.axis_index("s")
        w = c * NS + s
        r0 = s * ROWS_PER_TILE
        _fill_const(ones_v, EB, 128, 1.0)
        _fill_const(zero_v, ZB, 128, 0.0)
        _zero_slice(zero_v, acc, r0, zsem)
        plsc.subcore_barrier()
        for half in range(NH):
            pltpu.sync_copy(dst_hbm.at[w, half], idx_v)

            def body(j, carry):
                pltpu.async_copy(ones_v, acc.at[idx_v.at[j]], ssem, add=True)

                @pl.when(j >= LAG)
                def _():
                    pltpu.make_async_copy(ones_v, acc.at[idx_v.at[j]], ssem).wait()

                return carry

            lax.fori_loop(0, EJH, body, 0)
            for _ in range(LAG):
                pltpu.make_async_copy(ones_v, acc.at[idx_v.at[0]], ssem).wait()
        plsc.subcore_barrier()
        pltpu.sync_copy(acc.at[pl.ds(r0, ROWS_PER_TILE)],
                        out_hbm.at[c, pl.ds(r0, ROWS_PER_TILE)])

    return k(dst4)


def _prop_sc(tables, src4, dst4):
    C = len(tables)

    @functools.partial(
        pl.kernel,
        out_type=[jax.ShapeDtypeStruct((NC, NP, 128), jnp.float32) for _ in range(C)],
        mesh=_MESH,
        scratch_types=[
            pltpu.VMEM((EJH, EB), jnp.int32),
            pltpu.VMEM((EJH, EB), jnp.int32),
        ] + [pltpu.VMEM((EB, 128), jnp.float32) for _ in range(NLANE)]
        + [pltpu.SemaphoreType.DMA for _ in range(2 * NLANE + 1)]
        + [pltpu.VMEM_SHARED((NP, 128), jnp.float32)],
    )
    def k(*refs):
        h_hbms = refs[:C]
        src_hbm, dst_hbm = refs[C], refs[C + 1]
        outs = refs[C + 2:C + 2 + C]
        rest = refs[C + 2 + C:]
        src_v, dst_v = rest[0], rest[1]
        bufs = rest[2:2 + NLANE]
        gs = rest[2 + NLANE:2 + 2 * NLANE]
        ss = rest[2 + 2 * NLANE:2 + 3 * NLANE]
        zsem = rest[2 + 3 * NLANE]
        acc = rest[2 + 3 * NLANE + 1]
        c = lax.axis_index("c")
        s = lax.axis_index("s")
        w = c * NS + s
        r0 = s * ROWS_PER_TILE
        for ci in range(C):
            h = h_hbms[ci]
            _fill_const(bufs[0], ZB, 128, 0.0)
            _zero_slice(bufs[0], acc, r0, zsem)
            plsc.subcore_barrier()
            for half in range(NH):
                pltpu.sync_copy(src_hbm.at[w, half], src_v)
                pltpu.sync_copy(dst_hbm.at[w, half], dst_v)
                for k_ in range(NLANE):
                    pltpu.async_copy(h.at[src_v.at[k_]], bufs[k_], gs[k_])

                def body(gidx, carry, h=h):
                    base = gidx * NLANE
                    for k_ in range(NLANE):
                        j = base + k_
                        pltpu.make_async_copy(h.at[src_v.at[j]], bufs[k_], gs[k_]).wait()
                        pltpu.async_copy(bufs[k_], acc.at[dst_v.at[j]], ss[k_], add=True)
                    for k_ in range(NLANE):
                        j = base + k_

                        @pl.when(j + NLANE < EJH)
                        def _(j=j, k_=k_):
                            pltpu.make_async_copy(bufs[k_], acc.at[dst_v.at[j]], ss[k_]).wait()
                            pltpu.async_copy(h.at[src_v.at[j + NLANE]], bufs[k_], gs[k_])

                    return carry

                lax.fori_loop(0, EJH // NLANE, body, 0)
                for k_ in range(NLANE):
                    pltpu.make_async_copy(bufs[k_], acc.at[dst_v.at[k_]], ss[k_]).wait()
            plsc.subcore_barrier()
            pltpu.sync_copy(acc.at[pl.ds(r0, ROWS_PER_TILE)],
                            outs[ci].at[c, pl.ds(r0, ROWS_PER_TILE)])

    return k(*tables, src4, dst4)


BN = 400
GRID = (N // BN,)


def _rowspec(*lead):
    nl = len(lead)
    return pl.BlockSpec(tuple(lead) + (BN, 128),
                        lambda i, nl=nl: (0,) * nl + (i, 0))


def _fullspec(shape):
    nd = len(shape)
    return pl.BlockSpec(shape, lambda i, nd=nd: (0,) * nd)


def _layer_norm(z, g, b):
    mu = jnp.mean(z, axis=-1, keepdims=True)
    var = jnp.mean((z - mu) ** 2, axis=-1, keepdims=True)
    return (z - mu) * lax.rsqrt(var + 1e-5) * g + b


def _prep_tc(degp, x):
    def body(deg_ref, x_ref, dinv_ref, xp_ref):
        d = deg_ref[0] + deg_ref[1] + 1.0
        dvb = lax.rsqrt(d)
        dinv_ref[...] = dvb
        xp_ref[...] = x_ref[...] * dvb

    return pl.pallas_call(
        body,
        grid=GRID,
        in_specs=[_rowspec(NC), _rowspec()],
        out_specs=[_rowspec(), _rowspec()],
        out_shape=[jax.ShapeDtypeStruct((N, 128), jnp.float32)] * 2,
    )(degp, x)


def _layer1_mm2_tc(S1, xp, dinv, W1, b1, g1, bt1, W2):
    def body(S_ref, xp_ref, dv_ref, W1_ref, b_ref, g_ref, bt_ref, W2_ref,
             *out_refs):
        u = (S_ref[0] + S_ref[1] + xp_ref[...]) * dv_ref[...]
        z = jnp.dot(u.astype(jnp.bfloat16), W1_ref[...].astype(jnp.bfloat16),
                    preferred_element_type=jnp.float32)
        z = jax.nn.relu(z + b_ref[...])
        y = _layer_norm(z, g_ref[...], bt_ref[...])
        h2 = jnp.dot(y.astype(jnp.bfloat16), W2_ref[...].astype(jnp.bfloat16),
                     preferred_element_type=jnp.float32) * dv_ref[:, 0:1]
        for co in range(len(out_refs)):
            out_refs[co][...] = h2[:, co * 128:(co + 1) * 128]

    return pl.pallas_call(
        body,
        grid=GRID,
        in_specs=[_rowspec(NC), _rowspec(), _rowspec(),
                  _fullspec((128, 1024)), _fullspec((1, 1024)),
                  _fullspec((1, 1024)), _fullspec((1, 1024)),
                  _fullspec((1024, 512))],
        out_specs=[_rowspec()] * 4,
        out_shape=[jax.ShapeDtypeStruct((N, 128), jnp.float32)] * 4,
    )(S1, xp, dinv, W1, b1, g1, bt1, W2)


def _epilogue_tc(S_parts, h_chunks, dinv, b, g, bt, W_next):
    C = len(h_chunks)
    D = C * 128
    CO = 0 if W_next is None else W_next.shape[1] // 128

    def body(*refs):
        S_refs = refs[:C]
        h_refs = refs[C:2 * C]
        dv_ref, b_ref, g_ref, bt_ref = refs[2 * C:2 * C + 4]
        if W_next is None:
            out_refs = refs[2 * C + 4:]
        else:
            Wn_ref = refs[2 * C + 4]
            out_refs = refs[2 * C + 5:]
        dv = dv_ref[:, 0:1]
        parts = []
        for ci in range(C):
            zc = (S_refs[ci][0] + S_refs[ci][1] + h_refs[ci][...]) * dv
            parts.append(zc + b_ref[:, ci * 128:(ci + 1) * 128])
        z = jax.nn.relu(jnp.concatenate(parts, axis=1))
        y = _layer_norm(z, g_ref[...], bt_ref[...])
        if W_next is None:
            out_refs[0][...] = y
        else:
            hn = jnp.dot(y.astype(jnp.bfloat16),
                         Wn_ref[...].astype(jnp.bfloat16),
                         preferred_element_type=jnp.float32) * dv
            for co in range(CO):
                out_refs[co][...] = hn[:, co * 128:(co + 1) * 128]

    in_specs = ([_rowspec(NC)] * C + [_rowspec()] * C
                + [_rowspec(), _fullspec((1, D)), _fullspec((1, D)),
                   _fullspec((1, D))])
    ins = list(S_parts) + list(h_chunks) + [dinv, b, g, bt]
    if W_next is None:
        out_specs = [pl.BlockSpec((BN, D), lambda i: (i, 0))]
        out_shape = [jax.ShapeDtypeStruct((N, D), jnp.float32)]
    else:
        in_specs.append(_fullspec(W_next.shape))
        ins.append(W_next)
        out_specs = [_rowspec()] * CO
        out_shape = [jax.ShapeDtypeStruct((N, 128), jnp.float32)] * CO
    return pl.pallas_call(
        body,
        grid=GRID,
        in_specs=in_specs,
        out_specs=out_specs,
        out_shape=out_shape,
    )(*ins)


def kernel(data_batch, edge, W1, b1, g1, bt1, W2, b2, g2, bt2, W3, b3, g3, bt3):
    src4, dst4 = _pad_edges(edge)

    degp = _deg_sc(dst4)
    dinv, xp = _prep_tc(degp, data_batch)

    (S1,) = _prop_sc([xp], src4, dst4)
    h2_chunks = _layer1_mm2_tc(S1, xp, dinv, W1, b1.reshape(1, -1),
                               g1.reshape(1, -1), bt1.reshape(1, -1), W2)

    S2_parts = _prop_sc(list(h2_chunks), src4, dst4)
    h3_chunks = _epilogue_tc(S2_parts, h2_chunks, dinv, b2.reshape(1, -1),
                             g2.reshape(1, -1), bt2.reshape(1, -1), W3)

    S3_parts = _prop_sc(list(h3_chunks), src4, dst4)
    (x3,) = _epilogue_tc(S3_parts, h3_chunks, dinv, b3.reshape(1, -1),
                         g3.reshape(1, -1), bt3.reshape(1, -1), None)
    return x3
